# baseline (speedup 1.0000x reference)
import functools

import jax
import jax.numpy as jnp
from jax import lax
from jax.experimental import pallas as pl
from jax.experimental.pallas import tpu as pltpu
from jax.experimental.pallas import tpu_sc as plsc

N_FIELDS = 26
N_VOCAB = 100000
D = 32
B = 16384

NC = 2
NS = 16
LANES = 16

H0_LEN = 50048
H1_LEN = N_VOCAB - H0_LEN
CHUNK = 4096
N_CHUNKS = B // CHUNK
UNROLL = 8


def _scan_half(tb, base, length, fs, fb0, fb1, acc_v, semc0, semc1):
    cps = [None, None]
    fbs = [fb0, fb1]
    sems = [semc0, semc1]
    cps[0] = pltpu.make_async_copy(fs.at[pl.ds(0, CHUNK)], fb0, semc0)
    cps[0].start()
    for c in range(N_CHUNKS):
        if c + 1 < N_CHUNKS:
            cps[(c + 1) % 2] = pltpu.make_async_copy(
                fs.at[pl.ds((c + 1) * CHUNK, CHUNK)], fbs[(c + 1) % 2],
                sems[(c + 1) % 2])
            cps[(c + 1) % 2].start()
        cps[c % 2].wait()
        fb = fbs[c % 2]

        def samp_body(j, carry):
            for u in range(UNROLL):
                off = (j * UNROLL + u) * LANES
                v = fb[pl.ds(off, LANES)]
                vl = v - base
                m = (vl >= 0) & (vl < length)
                safe = jnp.where(m, vl, 0)
                g = plsc.load_gather(tb, [safe])
                gz = jnp.where(m, g, jnp.float32(0))
                a = c * CHUNK + off
                acc_v[pl.ds(a, LANES)] = acc_v[pl.ds(a, LANES)] + gz
            return carry
        lax.fori_loop(0, CHUNK // (LANES * UNROLL), samp_body, 0)


def _body(featT_hbm, tabT_hbm, out_hbm,
          tb0, tb1, fb0, fb1, acc_v, fs0, fs1,
          sem0, sem1, semf, semc0, semc1):
    d = lax.axis_index("c") * NS + lax.axis_index("s")
    is_stager = lax.axis_index("s") == 0

    zero = jnp.zeros((LANES,), jnp.float32)

    def zero_body(j, carry):
        acc_v[pl.ds(j * LANES, LANES)] = zero
        return carry
    lax.fori_loop(0, B // LANES, zero_body, 0)

    def tab_copy(i, half, buf, sem):
        if half == 0:
            return pltpu.make_async_copy(
                tabT_hbm.at[i, d, pl.ds(0, H0_LEN)], buf, sem)
        return pltpu.make_async_copy(
            tabT_hbm.at[i, d, pl.ds(H0_LEN, H1_LEN)], buf, sem)

    def feat_stage(i, fs):
        return pltpu.make_async_copy(featT_hbm.at[i], fs, semf)

    @pl.when(is_stager)
    def _():
        pltpu.sync_copy(featT_hbm.at[0], fs0)
    plsc.subcore_barrier()
    tab_copy(0, 0, tb0, sem0).start()

    def field_pair(fa, fs_a, fs_b, k):
        @pl.when(is_stager)
        def _():
            feat_stage(fa + 1, fs_b).start()
        tab_copy(fa, 1, tb1, sem1).start()
        tab_copy(fa, 0, tb0, sem0).wait()
        _scan_half(tb0, 0, H0_LEN, fs_a, fb0, fb1, acc_v, semc0, semc1)
        tab_copy(fa + 1, 0, tb0, sem0).start()
        tab_copy(fa, 1, tb1, sem1).wait()
        _scan_half(tb1, H0_LEN, H1_LEN,
                   fs_a, fb0, fb1, acc_v, semc0, semc1)

        @pl.when(is_stager)
        def _():
            feat_stage(fa + 1, fs_b).wait()
        plsc.subcore_barrier()

        @pl.when(is_stager & (fa + 2 < N_FIELDS))
        def _():
            feat_stage(fa + 2, fs_a).start()
        tab_copy(fa + 1, 1, tb1, sem1).start()
        tab_copy(fa + 1, 0, tb0, sem0).wait()
        _scan_half(tb0, 0, H0_LEN, fs_b, fb0, fb1, acc_v, semc0, semc1)

        @pl.when(fa + 2 < N_FIELDS)
        def _():
            tab_copy(fa + 2, 0, tb0, sem0).start()
        tab_copy(fa + 1, 1, tb1, sem1).wait()
        _scan_half(tb1, H0_LEN, H1_LEN,
                   fs_b, fb0, fb1, acc_v, semc0, semc1)

        @pl.when(is_stager & (fa + 2 < N_FIELDS))
        def _():
            feat_stage(fa + 2, fs_a).wait()
        plsc.subcore_barrier()
        return k

    lax.fori_loop(0, N_FIELDS // 2,
                  lambda k, c: field_pair(2 * k, fs0, fs1, c), 0)

    pltpu.sync_copy(acc_v, out_hbm.at[d])


@jax.jit
def _run(featT, tabT):
    mesh = plsc.VectorSubcoreMesh(core_axis_name="c", subcore_axis_name="s")
    f = functools.partial(
        pl.kernel,
        out_type=jax.ShapeDtypeStruct((D, B), jnp.float32),
        mesh=mesh,
        scratch_types=[
            pltpu.VMEM((H0_LEN,), jnp.float32),
            pltpu.VMEM((H1_LEN,), jnp.float32),
            pltpu.VMEM((CHUNK,), jnp.int32),
            pltpu.VMEM((CHUNK,), jnp.int32),
            pltpu.VMEM((B,), jnp.float32),
            pltpu.VMEM_SHARED((B,), jnp.int32),
            pltpu.VMEM_SHARED((B,), jnp.int32),
            pltpu.SemaphoreType.DMA,
            pltpu.SemaphoreType.DMA,
            pltpu.SemaphoreType.DMA,
            pltpu.SemaphoreType.DMA,
            pltpu.SemaphoreType.DMA,
        ],
        compiler_params=pltpu.CompilerParams(
            use_tc_tiling_on_sc=True, needs_layout_passes=False
        ),
    )(_body)
    return f(featT, tabT)


def kernel(features, tables):
    featT = features.astype(jnp.int32).T
    tabT = jnp.transpose(tables, (0, 2, 1))
    outT = _run(featT, tabT)
    return outT.T


# speedup vs baseline: 1.3407x; 1.3407x over previous
import functools

import jax
import jax.numpy as jnp
from jax import lax
from jax.experimental import pallas as pl
from jax.experimental.pallas import tpu as pltpu
from jax.experimental.pallas import tpu_sc as plsc

N_FIELDS = 26
N_VOCAB = 100000
D = 32
B = 16384

NC = 2
NS = 16
LANES = 16

FEAT_CHUNK = 8192
UNROLL = 16


def _body(featT_hbm, tabT_hbm, out_hbm, feat_v, tab_v, acc_v, sem_t):
    d = lax.axis_index("c") * NS + lax.axis_index("s")

    zero = jnp.zeros((LANES,), jnp.float32)

    def zero_body(j, carry):
        acc_v[pl.ds(j * LANES, LANES)] = zero
        return carry
    lax.fori_loop(0, B // LANES, zero_body, 0)

    def field_body(i, carry):
        cp = pltpu.make_async_copy(tabT_hbm.at[i, d, :], tab_v, sem_t)
        cp.start()
        cp.wait()

        def half_body(fb, c2):
            pltpu.sync_copy(featT_hbm.at[i, pl.ds(fb * FEAT_CHUNK, FEAT_CHUNK)],
                            feat_v)

            def samp_body(j, c3):
                for u in range(UNROLL):
                    off = j * LANES * UNROLL + u * LANES
                    v = feat_v[pl.ds(off, LANES)]
                    g = plsc.load_gather(tab_v, [v])
                    base = fb * FEAT_CHUNK + off
                    plsc.addupdate(acc_v.at[pl.ds(base, LANES)], g)
                return c3
            lax.fori_loop(0, FEAT_CHUNK // (LANES * UNROLL), samp_body, 0)
            return c2
        lax.fori_loop(0, B // FEAT_CHUNK, half_body, 0)
        return carry

    lax.fori_loop(0, N_FIELDS, field_body, 0)
    pltpu.sync_copy(acc_v, out_hbm.at[d])


@jax.jit
def _run(featT, tabT):
    mesh = plsc.VectorSubcoreMesh(core_axis_name="c", subcore_axis_name="s")
    f = functools.partial(
        pl.kernel,
        out_type=jax.ShapeDtypeStruct((D, B), jnp.float32),
        mesh=mesh,
        scratch_types=[
            pltpu.VMEM((FEAT_CHUNK,), jnp.int32),
            pltpu.VMEM((N_VOCAB,), jnp.float32),
            pltpu.VMEM((B,), jnp.float32),
            pltpu.SemaphoreType.DMA,
        ],
        compiler_params=pltpu.CompilerParams(
            use_tc_tiling_on_sc=True, needs_layout_passes=False
        ),
    )(_body)
    return f(featT, tabT)


def kernel(features, tables):
    featT = features.astype(jnp.int32).T
    tabT = jnp.transpose(tables, (0, 2, 1))
    outT = _run(featT, tabT)
    return outT.T


# speedup vs baseline: 1.6566x; 1.2357x over previous
import functools

import jax
import jax.numpy as jnp
from jax import lax
from jax.experimental import pallas as pl
from jax.experimental.pallas import tpu as pltpu
from jax.experimental.pallas import tpu_sc as plsc

N_FIELDS = 26
N_VOCAB = 100000
D = 32
B = 16384

NC = 2
NS = 16
LANES = 16

FEAT_CHUNK = 8192
UNROLL = 16


def _body(featT_hbm, tabT_hbm, out_hbm, feat_v, tab_v, acc_v, sem_t):
    d = lax.axis_index("c") * NS + lax.axis_index("s")

    zero = jnp.zeros((LANES,), jnp.float32)

    def zero_body(j, carry):
        acc_v[pl.ds(j * LANES, LANES)] = zero
        return carry
    lax.fori_loop(0, B // LANES, zero_body, 0)

    def field_body(i, carry):
        cp = pltpu.make_async_copy(tabT_hbm.at[i, d, :], tab_v, sem_t)
        cp.start()
        cp.wait()

        def half_body(fb, c2):
            pltpu.sync_copy(featT_hbm.at[i, pl.ds(fb * FEAT_CHUNK, FEAT_CHUNK)],
                            feat_v)

            @plsc.parallel_loop(0, FEAT_CHUNK // LANES, unroll=UNROLL)
            def samp_body(j):
                off = j * LANES
                v = feat_v[pl.ds(off, LANES)]
                g = plsc.load_gather(tab_v, [v])
                base = fb * FEAT_CHUNK + off
                plsc.addupdate(acc_v.at[pl.ds(base, LANES)], g)
            return c2
        lax.fori_loop(0, B // FEAT_CHUNK, half_body, 0)
        return carry

    lax.fori_loop(0, N_FIELDS, field_body, 0)
    pltpu.sync_copy(acc_v, out_hbm.at[d])


@jax.jit
def _run(featT, tabT):
    mesh = plsc.VectorSubcoreMesh(core_axis_name="c", subcore_axis_name="s")
    f = functools.partial(
        pl.kernel,
        out_type=jax.ShapeDtypeStruct((D, B), jnp.float32),
        mesh=mesh,
        scratch_types=[
            pltpu.VMEM((FEAT_CHUNK,), jnp.int32),
            pltpu.VMEM((N_VOCAB,), jnp.float32),
            pltpu.VMEM((B,), jnp.float32),
            pltpu.SemaphoreType.DMA,
        ],
        compiler_params=pltpu.CompilerParams(
            use_tc_tiling_on_sc=True, needs_layout_passes=False
        ),
    )(_body)
    return f(featT, tabT)


def kernel(features, tables):
    featT = features.astype(jnp.int32).T
    tabT = jnp.transpose(tables, (0, 2, 1))
    outT = _run(featT, tabT)
    return outT.T


# speedup vs baseline: 1.8379x; 1.1094x over previous
import functools

import jax
import jax.numpy as jnp
from jax import lax
from jax.experimental import pallas as pl
from jax.experimental.pallas import tpu as pltpu
from jax.experimental.pallas import tpu_sc as plsc

N_FIELDS = 26
N_VOCAB = 100000
D = 32
B = 16384

NC = 2
NS = 16
LANES = 16

FEAT_CHUNK = 4096
UNROLL = 16


def _body(featT_hbm, tabT_hbm, out_hbm, feat0_v, feat1_v, tab_v, acc_v,
          sem_t, sem_f0, sem_f1):
    d = lax.axis_index("c") * NS + lax.axis_index("s")

    zero = jnp.zeros((LANES,), jnp.float32)

    def zero_body(j, carry):
        acc_v[pl.ds(j * LANES, LANES)] = zero
        return carry
    lax.fori_loop(0, B // LANES, zero_body, 0)

    def feat_copy(i, fb, buf, sem):
        return pltpu.make_async_copy(
            featT_hbm.at[i, pl.ds(fb * FEAT_CHUNK, FEAT_CHUNK)], buf, sem)

    def scan_half(feat_v, fb):
        @plsc.parallel_loop(0, FEAT_CHUNK // LANES, unroll=UNROLL)
        def samp_body(j):
            off = j * LANES
            v = feat_v[pl.ds(off, LANES)]
            g = plsc.load_gather(tab_v, [v])
            base = fb * FEAT_CHUNK + off
            plsc.addupdate(acc_v.at[pl.ds(base, LANES)], g)

    def field_body(i, carry):
        bufs = [(feat0_v, sem_f0), (feat1_v, sem_f1)]
        cp = pltpu.make_async_copy(tabT_hbm.at[i, d, :], tab_v, sem_t)
        cp.start()
        feat_copy(i, 0, *bufs[0]).start()
        cp.wait()
        for c in range(B // FEAT_CHUNK):
            if c + 1 < B // FEAT_CHUNK:
                feat_copy(i, c + 1, *bufs[(c + 1) % 2]).start()
            feat_copy(i, c, *bufs[c % 2]).wait()
            scan_half(bufs[c % 2][0], c)
        return carry

    lax.fori_loop(0, N_FIELDS, field_body, 0)
    pltpu.sync_copy(acc_v, out_hbm.at[d])


@jax.jit
def _run(featT, tabT):
    mesh = plsc.VectorSubcoreMesh(core_axis_name="c", subcore_axis_name="s")
    f = functools.partial(
        pl.kernel,
        out_type=jax.ShapeDtypeStruct((D, B), jnp.float32),
        mesh=mesh,
        scratch_types=[
            pltpu.VMEM((FEAT_CHUNK,), jnp.int32),
            pltpu.VMEM((FEAT_CHUNK,), jnp.int32),
            pltpu.VMEM((N_VOCAB,), jnp.float32),
            pltpu.VMEM((B,), jnp.float32),
            pltpu.SemaphoreType.DMA,
            pltpu.SemaphoreType.DMA,
            pltpu.SemaphoreType.DMA,
        ],
        compiler_params=pltpu.CompilerParams(
            use_tc_tiling_on_sc=True, needs_layout_passes=False
        ),
    )(_body)
    return f(featT, tabT)


def kernel(features, tables):
    featT = features.astype(jnp.int32).T
    tabT = jnp.transpose(tables, (0, 2, 1))
    outT = _run(featT, tabT)
    return outT.T


# speedup vs baseline: 2.0561x; 1.1187x over previous
import functools

import jax
import jax.numpy as jnp
from jax import lax
from jax.experimental import pallas as pl
from jax.experimental.pallas import tpu as pltpu
from jax.experimental.pallas import tpu_sc as plsc

N_FIELDS = 26
N_VOCAB = 100000
D = 32
B = 16384

NC = 2
NS = 16
LANES = 16

H0_LEN = 50048
H1_LEN = N_VOCAB - H0_LEN
CHUNK = 4096
N_CHUNKS = B // CHUNK
UNROLL = 16


def _scan_half(tb, base, length, fs, fb0, fb1, acc_v, semc0, semc1):
    cps = [None, None]
    fbs = [fb0, fb1]
    sems = [semc0, semc1]
    cps[0] = pltpu.make_async_copy(fs.at[pl.ds(0, CHUNK)], fb0, semc0)
    cps[0].start()
    for c in range(N_CHUNKS):
        if c + 1 < N_CHUNKS:
            cps[(c + 1) % 2] = pltpu.make_async_copy(
                fs.at[pl.ds((c + 1) * CHUNK, CHUNK)], fbs[(c + 1) % 2],
                sems[(c + 1) % 2])
            cps[(c + 1) % 2].start()
        cps[c % 2].wait()
        fb = fbs[c % 2]

        @plsc.parallel_loop(0, CHUNK // LANES, unroll=UNROLL)
        def samp_body(j):
            off = j * LANES
            v = fb[pl.ds(off, LANES)]
            vl = v - base
            m = (vl >= 0) & (vl < length)
            safe = jnp.where(m, vl, 0)
            g = plsc.load_gather(tb, [safe])
            gz = jnp.where(m, g, jnp.float32(0))
            plsc.addupdate(acc_v.at[pl.ds(c * CHUNK + off, LANES)], gz)


def _body(featT_hbm, tabT_hbm, out_hbm,
          tb0, tb1, fb0, fb1, acc_v, fs0, fs1,
          sem0, sem1, semf, semc0, semc1):
    d = lax.axis_index("c") * NS + lax.axis_index("s")
    is_stager = lax.axis_index("s") == 0

    zero = jnp.zeros((LANES,), jnp.float32)

    def zero_body(j, carry):
        acc_v[pl.ds(j * LANES, LANES)] = zero
        return carry
    lax.fori_loop(0, B // LANES, zero_body, 0)

    def tab_copy(i, half, buf, sem):
        if half == 0:
            return pltpu.make_async_copy(
                tabT_hbm.at[i, d, pl.ds(0, H0_LEN)], buf, sem)
        return pltpu.make_async_copy(
            tabT_hbm.at[i, d, pl.ds(H0_LEN, H1_LEN)], buf, sem)

    def feat_stage(i, fs):
        return pltpu.make_async_copy(featT_hbm.at[i], fs, semf)

    @pl.when(is_stager)
    def _():
        pltpu.sync_copy(featT_hbm.at[0], fs0)
    plsc.subcore_barrier()
    tab_copy(0, 0, tb0, sem0).start()

    def field_pair(fa, fs_a, fs_b, k):
        @pl.when(is_stager)
        def _():
            feat_stage(fa + 1, fs_b).start()
        tab_copy(fa, 1, tb1, sem1).start()
        tab_copy(fa, 0, tb0, sem0).wait()
        _scan_half(tb0, 0, H0_LEN, fs_a, fb0, fb1, acc_v, semc0, semc1)
        tab_copy(fa + 1, 0, tb0, sem0).start()
        tab_copy(fa, 1, tb1, sem1).wait()
        _scan_half(tb1, H0_LEN, H1_LEN,
                   fs_a, fb0, fb1, acc_v, semc0, semc1)

        @pl.when(is_stager)
        def _():
            feat_stage(fa + 1, fs_b).wait()
        plsc.subcore_barrier()

        @pl.when(is_stager & (fa + 2 < N_FIELDS))
        def _():
            feat_stage(fa + 2, fs_a).start()
        tab_copy(fa + 1, 1, tb1, sem1).start()
        tab_copy(fa + 1, 0, tb0, sem0).wait()
        _scan_half(tb0, 0, H0_LEN, fs_b, fb0, fb1, acc_v, semc0, semc1)

        @pl.when(fa + 2 < N_FIELDS)
        def _():
            tab_copy(fa + 2, 0, tb0, sem0).start()
        tab_copy(fa + 1, 1, tb1, sem1).wait()
        _scan_half(tb1, H0_LEN, H1_LEN,
                   fs_b, fb0, fb1, acc_v, semc0, semc1)

        @pl.when(is_stager & (fa + 2 < N_FIELDS))
        def _():
            feat_stage(fa + 2, fs_a).wait()
        plsc.subcore_barrier()
        return k

    lax.fori_loop(0, N_FIELDS // 2,
                  lambda k, c: field_pair(2 * k, fs0, fs1, c), 0)

    pltpu.sync_copy(acc_v, out_hbm.at[d])


@jax.jit
def _run(featT, tabT):
    mesh = plsc.VectorSubcoreMesh(core_axis_name="c", subcore_axis_name="s")
    f = functools.partial(
        pl.kernel,
        out_type=jax.ShapeDtypeStruct((D, B), jnp.float32),
        mesh=mesh,
        scratch_types=[
            pltpu.VMEM((H0_LEN,), jnp.float32),
            pltpu.VMEM((H1_LEN,), jnp.float32),
            pltpu.VMEM((CHUNK,), jnp.int32),
            pltpu.VMEM((CHUNK,), jnp.int32),
            pltpu.VMEM((B,), jnp.float32),
            pltpu.VMEM_SHARED((B,), jnp.int32),
            pltpu.VMEM_SHARED((B,), jnp.int32),
            pltpu.SemaphoreType.DMA,
            pltpu.SemaphoreType.DMA,
            pltpu.SemaphoreType.DMA,
            pltpu.SemaphoreType.DMA,
            pltpu.SemaphoreType.DMA,
        ],
        compiler_params=pltpu.CompilerParams(
            use_tc_tiling_on_sc=True, needs_layout_passes=False
        ),
    )(_body)
    return f(featT, tabT)


def kernel(features, tables):
    featT = features.astype(jnp.int32).T
    tabT = jnp.transpose(tables, (0, 2, 1))
    outT = _run(featT, tabT)
    return outT.T
